# baseline (speedup 1.0000x reference)
import jax
import jax.numpy as jnp
from jax.experimental import pallas as pl
from jax.experimental.pallas import tpu as pltpu

_F32 = jnp.float32

_NGRAPH = 128
_NPG = 32
_EPG = 64
_MAXSET = 96
_ND = 128
_EIN = 16
_ED = 128
_TD = 64
_NPROP = 3
_NTIME = 2
_NPAIR = 64
_SINK_ITERS = 10
_SINK_TEMP = 0.1

_NUM_NODES = _NGRAPH * _NPG
_NUM_EDGES = _NGRAPH * _EPG

_BN = 2048
_BE = 4096
_BPAIR = 32
_G = _NUM_NODES // _BN
_SN = 128
_SE = 256
_NSUB = _BN // _SN

_CH = 136


def _mm(a, b):
    return jax.lax.dot_general(a, b, (((a.ndim - 1,), (0,)), ((), ())),
                               preferred_element_type=_F32)


def _mmT(a, b):
    return jax.lax.dot_general(a, b, (((0,), (0,)), ((), ())),
                               preferred_element_type=_F32)


def _main_kernel(nf_r, ef_r, lf_r, lt_r,
                 wn_r, bn_r, we_r, be_r,
                 wc1_r, bc1_r, wc2_r, bc2_r,
                 wm1_r, bm1_r, wm2_r, bm2_r,
                 wr1_r, br1_r, wr2_r, br2_r,
                 wu1_r, bu1_r, wu2_r, bu2_r,
                 w1_r, b1_r, w2_r, b2_r,
                 score_r):
    enc_n = jnp.maximum(_mm(nf_r[...], wn_r[...]) + bn_r[...], 0.0)
    enc_e = jnp.maximum(_mm(ef_r[...], we_r[...]) + be_r[...], 0.0)

    lf = lf_r[0]
    lt = lt_r[0]
    iota = jax.lax.broadcasted_iota(jnp.int32, (_SN, _SE), 0)
    ohf = [(iota == (lf[:, s * _SE:(s + 1) * _SE] - s * _SN)).astype(_F32)
           for s in range(_NSUB)]
    oht = [(iota == (lt[:, s * _SE:(s + 1) * _SE] - s * _SN)).astype(_F32)
           for s in range(_NSUB)]

    wc1 = wc1_r[...]
    wm1 = wm1_r[...]
    wr1 = wr1_r[...]
    wu1 = wu1_r[...]
    w1 = w1_r[...]
    w2 = w2_r[...]

    def gather(x):
        fs, ts = [], []
        for s in range(_NSUB):
            xs = x[s * _SN:(s + 1) * _SN]
            fs.append(_mmT(ohf[s], xs))
            ts.append(_mmT(oht[s], xs))
        return jnp.concatenate(fs, axis=0), jnp.concatenate(ts, axis=0)

    def scatter(m, r):
        outs = []
        for s in range(_NSUB):
            sl = slice(s * _SE, (s + 1) * _SE)
            outs.append(_mm(oht[s], m[sl]) + _mm(ohf[s], r[sl]))
        return jnp.concatenate(outs, axis=0)

    def msgs(src, dst, ee):
        hm = jnp.maximum(_mm(src, wm1[:_ND]) + _mm(dst, wm1[_ND:2 * _ND])
                         + _mm(ee, wm1[2 * _ND:]) + bm1_r[...], 0.0)
        m = _mm(hm, wm2_r[...]) + bm2_r[...]
        hr = jnp.maximum(_mm(dst, wr1[:_ND]) + _mm(src, wr1[_ND:2 * _ND])
                         + _mm(ee, wr1[2 * _ND:]) + br1_r[...], 0.0)
        r = _mm(hr, wr2_r[...]) + br2_r[...]
        return m, r

    def dg(a, b, ca, cb):
        return jax.lax.dot_general(a, b, (((ca,), (cb,)), ((0,), (0,))),
                                   preferred_element_type=_F32)

    def tmlp(x):
        h = jnp.maximum(jax.lax.dot_general(x, w1, (((2,), (0,)), ((), ())),
                                            preferred_element_type=_F32) + b1_r[...], 0.0)
        return jax.lax.dot_general(h, w2, (((2,), (0,)), ((), ())),
                                   preferred_element_type=_F32) + b2_r[...]

    inter = jnp.zeros((_BN, _ED), _F32)
    for t in range(_NTIME):
        ne = enc_n
        for _ in range(_NPROP):
            h = jnp.maximum(_mm(ne, wc1[:_ND]) + _mm(inter, wc1[_ND:]) + bc1_r[...], 0.0)
            comb = _mm(h, wc2_r[...]) + bc2_r[...]
            src, dst = gather(comb)
            msg, rmsg = msgs(src, dst, enc_e)
            agg = scatter(msg, rmsg)
            hu = jnp.maximum(_mm(comb, wu1[:_ND]) + _mm(agg, wu1[_ND:]) + bu1_r[...], 0.0)
            ne = _mm(hu, wu2_r[...]) + bu2_r[...]

        src, dst = gather(ne)
        fwd, bwd = msgs(src, dst, enc_e)
        ee2 = (fwd + bwd).reshape(_BPAIR, 2 * _EPG, _ED)
        eq = ee2[:, :_EPG, :]
        ec = ee2[:, _EPG:, :]

        tq = tmlp(eq)
        tc = tmlp(ec)
        ztd = jnp.zeros((_BPAIR, _MAXSET - _EPG, _TD), _F32)
        mq = jnp.concatenate([tq, ztd], axis=1)
        mc = jnp.concatenate([tc, ztd], axis=1)

        la = dg(mq, mc, 2, 2) / jnp.float32(_SINK_TEMP)
        for _ in range(_SINK_ITERS):
            m2 = jnp.max(la, axis=2, keepdims=True)
            la = la - (m2 + jnp.log(jnp.sum(jnp.exp(la - m2), axis=2, keepdims=True)))
            m1 = jnp.max(la, axis=1, keepdims=True)
            la = la - (m1 + jnp.log(jnp.sum(jnp.exp(la - m1), axis=1, keepdims=True)))
        tmat = jnp.exp(la)

        if t < _NTIME - 1:
            qfc = dg(tmat[:, :_NPG, :_EPG], ec, 2, 1)
            cfq = dg(tmat[:, :_EPG, :_NPG], eq, 1, 1)
            inter = jnp.concatenate([qfc, cfq], axis=1).reshape(_BN, _ED)
        else:
            tmc = dg(tmat[:, :, :_EPG], tc, 2, 1)
            s2 = jnp.sum(jnp.abs(mq - tmc), axis=2)
            s1 = jnp.sum(s2, axis=1, keepdims=True)
            score_r[...] = jnp.broadcast_to(-s1, (_BPAIR, 128))


def _full(shape):
    nd = len(shape)
    return pl.BlockSpec(shape, lambda i: (0,) * nd)


def kernel(node_features, edge_features, from_idx, to_idx, graph_idx, params):
    p = params
    lf = jnp.mod(from_idx, _BN).astype(jnp.int32).reshape(_G, 1, _BE)
    lt = jnp.mod(to_idx, _BN).astype(jnp.int32).reshape(_G, 1, _BE)

    wc1 = jnp.pad(p["comb_W1"], ((0, 0), (0, _CH - 129)))
    bc1 = jnp.pad(p["comb_b1"], (0, _CH - 129)).reshape(1, _CH)
    wc2 = jnp.pad(p["comb_W2"], ((0, _CH - 129), (0, 0)))

    def b2(x):
        return x.reshape(1, -1)

    weights = (
        p["enc_node_W"], b2(p["enc_node_b"]), p["enc_edge_W"], b2(p["enc_edge_b"]),
        wc1, bc1, wc2, b2(p["comb_b2"]),
        p["msg_W1"], b2(p["msg_b1"]), p["msg_W2"], b2(p["msg_b2"]),
        p["rmsg_W1"], b2(p["rmsg_b1"]), p["rmsg_W2"], b2(p["rmsg_b2"]),
        p["upd_W1"], b2(p["upd_b1"]), p["upd_W2"], b2(p["upd_b2"]),
        p["t1_W"], b2(p["t1_b"]), p["t2_W"], b2(p["t2_b"]),
    )

    scores = pl.pallas_call(
        _main_kernel,
        grid=(_G,),
        in_specs=[
            pl.BlockSpec((_BN, _ND), lambda i: (i, 0)),
            pl.BlockSpec((_BE, _EIN), lambda i: (i, 0)),
            pl.BlockSpec((1, 1, _BE), lambda i: (i, 0, 0)),
            pl.BlockSpec((1, 1, _BE), lambda i: (i, 0, 0)),
        ] + [_full(w.shape) for w in weights],
        out_specs=pl.BlockSpec((_BPAIR, 128), lambda i: (i, 0)),
        out_shape=jax.ShapeDtypeStruct((_NPAIR, 128), _F32),
        compiler_params=pltpu.CompilerParams(
            dimension_semantics=("parallel",),
            vmem_limit_bytes=100 * 1024 * 1024),
    )(node_features, edge_features, lf, lt, *weights)

    return scores[:, 0]


# speedup vs baseline: 1.1942x; 1.1942x over previous
import jax
import jax.numpy as jnp
from jax.experimental import pallas as pl
from jax.experimental.pallas import tpu as pltpu

_F32 = jnp.float32

_NGRAPH = 128
_NPG = 32
_EPG = 64
_MAXSET = 96
_ND = 128
_EIN = 16
_ED = 128
_TD = 64
_NPROP = 3
_NTIME = 2
_NPAIR = 64
_SINK_ITERS = 10
_SINK_TEMP = 0.1

_NUM_NODES = _NGRAPH * _NPG
_NUM_EDGES = _NGRAPH * _EPG

_BN = 1024
_BE = 2048
_BPAIR = 16
_G = _NUM_NODES // _BN
_SN = 128
_SE = 256
_NSUB = _BN // _SN

_CH = 136


def _mm(a, b):
    return jax.lax.dot_general(a, b, (((a.ndim - 1,), (0,)), ((), ())),
                               preferred_element_type=_F32)


def _mmT(a, b):
    return jax.lax.dot_general(a, b, (((0,), (0,)), ((), ())),
                               preferred_element_type=_F32)


def _main_kernel(nf_r, ef_r, lf_r, lt_r,
                 wn_r, bn_r, we_r, be_r,
                 wc1_r, bc1_r, wc2_r, bc2_r,
                 wm1_r, bm1_r, wm2_r, bm2_r,
                 wr1_r, br1_r, wr2_r, br2_r,
                 wu1_r, bu1_r, wu2_r, bu2_r,
                 w1_r, b1_r, w2_r, b2_r,
                 score_r):
    enc_n = jnp.maximum(_mm(nf_r[...], wn_r[...]) + bn_r[...], 0.0)
    enc_e = jnp.maximum(_mm(ef_r[...], we_r[...]) + be_r[...], 0.0)

    lf = lf_r[0]
    lt = lt_r[0]
    iota = jax.lax.broadcasted_iota(jnp.int32, (_SN, _SE), 0)
    ohf = [(iota == (lf[:, s * _SE:(s + 1) * _SE] - s * _SN)).astype(_F32)
           for s in range(_NSUB)]
    oht = [(iota == (lt[:, s * _SE:(s + 1) * _SE] - s * _SN)).astype(_F32)
           for s in range(_NSUB)]

    wc1 = wc1_r[...]
    wm1 = wm1_r[...]
    wr1 = wr1_r[...]
    wu1 = wu1_r[...]
    w1 = w1_r[...]
    w2 = w2_r[...]

    def gather(x):
        fs, ts = [], []
        for s in range(_NSUB):
            xs = x[s * _SN:(s + 1) * _SN]
            fs.append(_mmT(ohf[s], xs))
            ts.append(_mmT(oht[s], xs))
        return jnp.concatenate(fs, axis=0), jnp.concatenate(ts, axis=0)

    def scatter(m, r):
        outs = []
        for s in range(_NSUB):
            sl = slice(s * _SE, (s + 1) * _SE)
            outs.append(_mm(oht[s], m[sl]) + _mm(ohf[s], r[sl]))
        return jnp.concatenate(outs, axis=0)

    def msgs(src, dst, ee):
        hm = jnp.maximum(_mm(src, wm1[:_ND]) + _mm(dst, wm1[_ND:2 * _ND])
                         + _mm(ee, wm1[2 * _ND:]) + bm1_r[...], 0.0)
        m = _mm(hm, wm2_r[...]) + bm2_r[...]
        hr = jnp.maximum(_mm(dst, wr1[:_ND]) + _mm(src, wr1[_ND:2 * _ND])
                         + _mm(ee, wr1[2 * _ND:]) + br1_r[...], 0.0)
        r = _mm(hr, wr2_r[...]) + br2_r[...]
        return m, r

    def dg(a, b, ca, cb):
        return jax.lax.dot_general(a, b, (((ca,), (cb,)), ((0,), (0,))),
                                   preferred_element_type=_F32)

    def tmlp(x):
        h = jnp.maximum(jax.lax.dot_general(x, w1, (((2,), (0,)), ((), ())),
                                            preferred_element_type=_F32) + b1_r[...], 0.0)
        return jax.lax.dot_general(h, w2, (((2,), (0,)), ((), ())),
                                   preferred_element_type=_F32) + b2_r[...]

    inter = jnp.zeros((_BN, _ED), _F32)
    for t in range(_NTIME):
        ne = enc_n
        for _ in range(_NPROP):
            h = jnp.maximum(_mm(ne, wc1[:_ND]) + _mm(inter, wc1[_ND:]) + bc1_r[...], 0.0)
            comb = _mm(h, wc2_r[...]) + bc2_r[...]
            src, dst = gather(comb)
            msg, rmsg = msgs(src, dst, enc_e)
            agg = scatter(msg, rmsg)
            hu = jnp.maximum(_mm(comb, wu1[:_ND]) + _mm(agg, wu1[_ND:]) + bu1_r[...], 0.0)
            ne = _mm(hu, wu2_r[...]) + bu2_r[...]

        src, dst = gather(ne)
        fwd, bwd = msgs(src, dst, enc_e)
        ee2 = (fwd + bwd).reshape(_BPAIR, 2 * _EPG, _ED)
        eq = ee2[:, :_EPG, :]
        ec = ee2[:, _EPG:, :]

        tq = tmlp(eq)
        tc = tmlp(ec)
        ztd = jnp.zeros((_BPAIR, _MAXSET - _EPG, _TD), _F32)
        mq = jnp.concatenate([tq, ztd], axis=1)
        mc = jnp.concatenate([tc, ztd], axis=1)

        la = dg(mq, mc, 2, 2) / jnp.float32(_SINK_TEMP)
        for _ in range(_SINK_ITERS):
            m2 = jnp.max(la, axis=2, keepdims=True)
            la = la - (m2 + jnp.log(jnp.sum(jnp.exp(la - m2), axis=2, keepdims=True)))
            m1 = jnp.max(la, axis=1, keepdims=True)
            la = la - (m1 + jnp.log(jnp.sum(jnp.exp(la - m1), axis=1, keepdims=True)))
        tmat = jnp.exp(la)

        if t < _NTIME - 1:
            qfc = dg(tmat[:, :_NPG, :_EPG], ec, 2, 1)
            cfq = dg(tmat[:, :_EPG, :_NPG], eq, 1, 1)
            inter = jnp.concatenate([qfc, cfq], axis=1).reshape(_BN, _ED)
        else:
            tmc = dg(tmat[:, :, :_EPG], tc, 2, 1)
            s2 = jnp.sum(jnp.abs(mq - tmc), axis=2)
            s1 = jnp.sum(s2, axis=1, keepdims=True)
            score_r[...] = jnp.broadcast_to(-s1, (_BPAIR, 128))


def _full(shape):
    nd = len(shape)
    return pl.BlockSpec(shape, lambda i: (0,) * nd)


def kernel(node_features, edge_features, from_idx, to_idx, graph_idx, params):
    p = params
    lf = jnp.mod(from_idx, _BN).astype(jnp.int32).reshape(_G, 1, _BE)
    lt = jnp.mod(to_idx, _BN).astype(jnp.int32).reshape(_G, 1, _BE)

    wc1 = jnp.pad(p["comb_W1"], ((0, 0), (0, _CH - 129)))
    bc1 = jnp.pad(p["comb_b1"], (0, _CH - 129)).reshape(1, _CH)
    wc2 = jnp.pad(p["comb_W2"], ((0, _CH - 129), (0, 0)))

    def b2(x):
        return x.reshape(1, -1)

    weights = (
        p["enc_node_W"], b2(p["enc_node_b"]), p["enc_edge_W"], b2(p["enc_edge_b"]),
        wc1, bc1, wc2, b2(p["comb_b2"]),
        p["msg_W1"], b2(p["msg_b1"]), p["msg_W2"], b2(p["msg_b2"]),
        p["rmsg_W1"], b2(p["rmsg_b1"]), p["rmsg_W2"], b2(p["rmsg_b2"]),
        p["upd_W1"], b2(p["upd_b1"]), p["upd_W2"], b2(p["upd_b2"]),
        p["t1_W"], b2(p["t1_b"]), p["t2_W"], b2(p["t2_b"]),
    )

    scores = pl.pallas_call(
        _main_kernel,
        grid=(_G,),
        in_specs=[
            pl.BlockSpec((_BN, _ND), lambda i: (i, 0)),
            pl.BlockSpec((_BE, _EIN), lambda i: (i, 0)),
            pl.BlockSpec((1, 1, _BE), lambda i: (i, 0, 0)),
            pl.BlockSpec((1, 1, _BE), lambda i: (i, 0, 0)),
        ] + [_full(w.shape) for w in weights],
        out_specs=pl.BlockSpec((_BPAIR, 128), lambda i: (i, 0)),
        out_shape=jax.ShapeDtypeStruct((_NPAIR, 128), _F32),
        compiler_params=pltpu.CompilerParams(
            dimension_semantics=("arbitrary",),
            vmem_limit_bytes=100 * 1024 * 1024),
    )(node_features, edge_features, lf, lt, *weights)

    return scores[:, 0]


# speedup vs baseline: 1.2196x; 1.0213x over previous
import jax
import jax.numpy as jnp
from jax.experimental import pallas as pl
from jax.experimental.pallas import tpu as pltpu

_F32 = jnp.float32

_NGRAPH = 128
_NPG = 32
_EPG = 64
_MAXSET = 96
_ND = 128
_EIN = 16
_ED = 128
_TD = 64
_NPROP = 3
_NTIME = 2
_NPAIR = 64
_SINK_ITERS = 10
_SINK_TEMP = 0.1

_NUM_NODES = _NGRAPH * _NPG
_NUM_EDGES = _NGRAPH * _EPG

_BN = 1024
_BE = 2048
_BPAIR = 16
_G = _NUM_NODES // _BN
_SN = 128
_SE = 256
_NSUB = _BN // _SN

_CH = 136


def _mm(a, b):
    return jax.lax.dot_general(a, b, (((a.ndim - 1,), (0,)), ((), ())),
                               preferred_element_type=_F32)


def _mmT(a, b):
    return jax.lax.dot_general(a, b, (((0,), (0,)), ((), ())),
                               preferred_element_type=_F32)


def _main_kernel(nf_r, ef_r, lf_r, lt_r,
                 wn_r, bn_r, we_r, be_r,
                 wc1_r, bc1_r, wc2_r, bc2_r,
                 wm1_r, bm1_r, wm2_r, bm2_r,
                 wr1_r, br1_r, wr2_r, br2_r,
                 wu1_r, bu1_r, wu2_r, bu2_r,
                 w1_r, b1_r, w2_r, b2_r,
                 score_r):
    enc_n = jnp.maximum(_mm(nf_r[...], wn_r[...]) + bn_r[...], 0.0)
    enc_e = jnp.maximum(_mm(ef_r[...], we_r[...]) + be_r[...], 0.0)

    lf = jnp.bitwise_and(lf_r[0], _BN - 1)
    lt = jnp.bitwise_and(lt_r[0], _BN - 1)
    iota = jax.lax.broadcasted_iota(jnp.int32, (_SN, _SE), 0)
    ohf = [(iota == (lf[:, s * _SE:(s + 1) * _SE] - s * _SN)).astype(_F32)
           for s in range(_NSUB)]
    oht = [(iota == (lt[:, s * _SE:(s + 1) * _SE] - s * _SN)).astype(_F32)
           for s in range(_NSUB)]

    wc1 = wc1_r[...]
    wm1 = wm1_r[...]
    wr1 = wr1_r[...]
    wu1 = wu1_r[...]
    w1 = w1_r[...]
    w2 = w2_r[...]

    def gather(x):
        fs, ts = [], []
        for s in range(_NSUB):
            xs = x[s * _SN:(s + 1) * _SN]
            fs.append(_mmT(ohf[s], xs))
            ts.append(_mmT(oht[s], xs))
        return jnp.concatenate(fs, axis=0), jnp.concatenate(ts, axis=0)

    def scatter(m, r):
        outs = []
        for s in range(_NSUB):
            sl = slice(s * _SE, (s + 1) * _SE)
            outs.append(_mm(oht[s], m[sl]) + _mm(ohf[s], r[sl]))
        return jnp.concatenate(outs, axis=0)

    def msgs(src, dst, ee):
        hm = jnp.maximum(_mm(src, wm1[:_ND]) + _mm(dst, wm1[_ND:2 * _ND])
                         + _mm(ee, wm1[2 * _ND:]) + bm1_r[...], 0.0)
        m = _mm(hm, wm2_r[...]) + bm2_r[...]
        hr = jnp.maximum(_mm(dst, wr1[:_ND]) + _mm(src, wr1[_ND:2 * _ND])
                         + _mm(ee, wr1[2 * _ND:]) + br1_r[...], 0.0)
        r = _mm(hr, wr2_r[...]) + br2_r[...]
        return m, r

    def dg(a, b, ca, cb):
        return jax.lax.dot_general(a, b, (((ca,), (cb,)), ((0,), (0,))),
                                   preferred_element_type=_F32)

    def tmlp(x):
        h = jnp.maximum(jax.lax.dot_general(x, w1, (((2,), (0,)), ((), ())),
                                            preferred_element_type=_F32) + b1_r[...], 0.0)
        return jax.lax.dot_general(h, w2, (((2,), (0,)), ((), ())),
                                   preferred_element_type=_F32) + b2_r[...]

    inter = jnp.zeros((_BN, _ED), _F32)
    for t in range(_NTIME):
        ne = enc_n
        for _ in range(_NPROP):
            h = jnp.maximum(_mm(ne, wc1[:_ND]) + _mm(inter, wc1[_ND:]) + bc1_r[...], 0.0)
            comb = _mm(h, wc2_r[...]) + bc2_r[...]
            src, dst = gather(comb)
            msg, rmsg = msgs(src, dst, enc_e)
            agg = scatter(msg, rmsg)
            hu = jnp.maximum(_mm(comb, wu1[:_ND]) + _mm(agg, wu1[_ND:]) + bu1_r[...], 0.0)
            ne = _mm(hu, wu2_r[...]) + bu2_r[...]

        src, dst = gather(ne)
        fwd, bwd = msgs(src, dst, enc_e)
        ee2 = (fwd + bwd).reshape(_BPAIR, 2 * _EPG, _ED)
        eq = ee2[:, :_EPG, :]
        ec = ee2[:, _EPG:, :]

        tq = tmlp(eq)
        tc = tmlp(ec)
        ztd = jnp.zeros((_BPAIR, _MAXSET - _EPG, _TD), _F32)
        mq = jnp.concatenate([tq, ztd], axis=1)
        mc = jnp.concatenate([tc, ztd], axis=1)

        la = dg(mq, mc, 2, 2) / jnp.float32(_SINK_TEMP)
        for _ in range(_SINK_ITERS):
            m2 = jnp.max(la, axis=2, keepdims=True)
            la = la - (m2 + jnp.log(jnp.sum(jnp.exp(la - m2), axis=2, keepdims=True)))
            m1 = jnp.max(la, axis=1, keepdims=True)
            la = la - (m1 + jnp.log(jnp.sum(jnp.exp(la - m1), axis=1, keepdims=True)))
        tmat = jnp.exp(la)

        if t < _NTIME - 1:
            qfc = dg(tmat[:, :_NPG, :_EPG], ec, 2, 1)
            cfq = dg(tmat[:, :_EPG, :_NPG], eq, 1, 1)
            inter = jnp.concatenate([qfc, cfq], axis=1).reshape(_BN, _ED)
        else:
            tmc = dg(tmat[:, :, :_EPG], tc, 2, 1)
            s2 = jnp.sum(jnp.abs(mq - tmc), axis=2)
            s1 = jnp.sum(s2, axis=1, keepdims=True)
            score_r[...] = jnp.broadcast_to(-s1, (_BPAIR, 128))


def _full(shape):
    nd = len(shape)
    return pl.BlockSpec(shape, lambda i: (0,) * nd)


def kernel(node_features, edge_features, from_idx, to_idx, graph_idx, params):
    p = params
    lf = from_idx.astype(jnp.int32).reshape(_G, 1, _BE)
    lt = to_idx.astype(jnp.int32).reshape(_G, 1, _BE)

    wc1 = jnp.pad(p["comb_W1"], ((0, 0), (0, _CH - 129)))
    bc1 = jnp.pad(p["comb_b1"], (0, _CH - 129)).reshape(1, _CH)
    wc2 = jnp.pad(p["comb_W2"], ((0, _CH - 129), (0, 0)))

    def b2(x):
        return x.reshape(1, -1)

    weights = (
        p["enc_node_W"], b2(p["enc_node_b"]), p["enc_edge_W"], b2(p["enc_edge_b"]),
        wc1, bc1, wc2, b2(p["comb_b2"]),
        p["msg_W1"], b2(p["msg_b1"]), p["msg_W2"], b2(p["msg_b2"]),
        p["rmsg_W1"], b2(p["rmsg_b1"]), p["rmsg_W2"], b2(p["rmsg_b2"]),
        p["upd_W1"], b2(p["upd_b1"]), p["upd_W2"], b2(p["upd_b2"]),
        p["t1_W"], b2(p["t1_b"]), p["t2_W"], b2(p["t2_b"]),
    )

    scores = pl.pallas_call(
        _main_kernel,
        grid=(_G,),
        in_specs=[
            pl.BlockSpec((_BN, _ND), lambda i: (i, 0)),
            pl.BlockSpec((_BE, _EIN), lambda i: (i, 0)),
            pl.BlockSpec((1, 1, _BE), lambda i: (i, 0, 0)),
            pl.BlockSpec((1, 1, _BE), lambda i: (i, 0, 0)),
        ] + [_full(w.shape) for w in weights],
        out_specs=pl.BlockSpec((_BPAIR, 128), lambda i: (i, 0)),
        out_shape=jax.ShapeDtypeStruct((_NPAIR, 128), _F32),
        compiler_params=pltpu.CompilerParams(
            dimension_semantics=("arbitrary",),
            vmem_limit_bytes=100 * 1024 * 1024),
    )(node_features, edge_features, lf, lt, *weights)

    return scores[:, 0]


# speedup vs baseline: 1.2450x; 1.0208x over previous
import jax
import jax.numpy as jnp
from jax.experimental import pallas as pl
from jax.experimental.pallas import tpu as pltpu

_F32 = jnp.float32

_NGRAPH = 128
_NPG = 32
_EPG = 64
_MAXSET = 96
_ND = 128
_EIN = 16
_ED = 128
_TD = 64
_NPROP = 3
_NTIME = 2
_NPAIR = 64
_SINK_ITERS = 10
_SINK_TEMP = 0.1

_NUM_NODES = _NGRAPH * _NPG
_NUM_EDGES = _NGRAPH * _EPG

_BN = 1024
_BE = 2048
_BPAIR = 16
_G = _NUM_NODES // _BN
_SN = 128
_SE = 256
_NSUB = _BN // _SN

_CH = 136


def _mm(a, b):
    return jax.lax.dot_general(a, b, (((a.ndim - 1,), (0,)), ((), ())),
                               preferred_element_type=_F32)


def _mmT(a, b):
    return jax.lax.dot_general(a, b, (((0,), (0,)), ((), ())),
                               preferred_element_type=_F32)


def _main_kernel(nf_r, ef_r, lf_r, lt_r,
                 wn_r, bn_r, we_r, be_r,
                 wc1_r, bc1_r, wc2_r, bc2_r,
                 wm1_r, bm1_r, wm2_r, bm2_r,
                 wr1_r, br1_r, wr2_r, br2_r,
                 wu1_r, bu1_r, wu2_r, bu2_r,
                 w1_r, b1_r, w2_r, b2_r,
                 score_r):
    enc_n = jnp.maximum(_mm(nf_r[...], wn_r[...]) + bn_r[...], 0.0)
    enc_e = jnp.maximum(_mm(ef_r[...], we_r[...]) + be_r[...], 0.0)

    lf = jnp.bitwise_and(lf_r[0], _BN - 1)
    lt = jnp.bitwise_and(lt_r[0], _BN - 1)
    iota = jax.lax.broadcasted_iota(jnp.int32, (_SN, _SE), 0)
    ohf = [(iota == (lf[:, s * _SE:(s + 1) * _SE] - s * _SN)).astype(_F32)
           for s in range(_NSUB)]
    oht = [(iota == (lt[:, s * _SE:(s + 1) * _SE] - s * _SN)).astype(_F32)
           for s in range(_NSUB)]

    wc1 = wc1_r[...]
    wm1 = wm1_r[...]
    wr1 = wr1_r[...]
    wu1 = wu1_r[...]
    w1 = w1_r[...]
    w2 = w2_r[...]

    def gather(x):
        fs, ts = [], []
        for s in range(_NSUB):
            xs = x[s * _SN:(s + 1) * _SN]
            fs.append(_mmT(ohf[s], xs))
            ts.append(_mmT(oht[s], xs))
        return jnp.concatenate(fs, axis=0), jnp.concatenate(ts, axis=0)

    def scatter(m, r):
        outs = []
        for s in range(_NSUB):
            sl = slice(s * _SE, (s + 1) * _SE)
            outs.append(_mm(oht[s], m[sl]) + _mm(ohf[s], r[sl]))
        return jnp.concatenate(outs, axis=0)

    def msgs(src, dst, ee):
        hm = jnp.maximum(_mm(src, wm1[:_ND]) + _mm(dst, wm1[_ND:2 * _ND])
                         + _mm(ee, wm1[2 * _ND:]) + bm1_r[...], 0.0)
        m = _mm(hm, wm2_r[...]) + bm2_r[...]
        hr = jnp.maximum(_mm(dst, wr1[:_ND]) + _mm(src, wr1[_ND:2 * _ND])
                         + _mm(ee, wr1[2 * _ND:]) + br1_r[...], 0.0)
        r = _mm(hr, wr2_r[...]) + br2_r[...]
        return m, r

    def dg(a, b, ca, cb):
        return jax.lax.dot_general(a, b, (((ca,), (cb,)), ((0,), (0,))),
                                   preferred_element_type=_F32)

    inter = jnp.zeros((_BN, _ED), _F32)
    for t in range(_NTIME):
        ne = enc_n
        for _ in range(_NPROP):
            h = jnp.maximum(_mm(ne, wc1[:_ND]) + _mm(inter, wc1[_ND:]) + bc1_r[...], 0.0)
            comb = _mm(h, wc2_r[...]) + bc2_r[...]
            src, dst = gather(comb)
            msg, rmsg = msgs(src, dst, enc_e)
            agg = scatter(msg, rmsg)
            hu = jnp.maximum(_mm(comb, wu1[:_ND]) + _mm(agg, wu1[_ND:]) + bu1_r[...], 0.0)
            ne = _mm(hu, wu2_r[...]) + bu2_r[...]

        src, dst = gather(ne)
        fwd, bwd = msgs(src, dst, enc_e)
        eeflat = fwd + bwd
        ee2 = eeflat.reshape(_BPAIR, 2 * _EPG, _ED)
        eq = ee2[:, :_EPG, :]
        ec = ee2[:, _EPG:, :]

        tflat = _mm(jnp.maximum(_mm(eeflat, w1) + b1_r[...], 0.0), w2) + b2_r[...]
        t3 = tflat.reshape(_BPAIR, 2 * _EPG, _TD)
        tq = t3[:, :_EPG, :]
        tc = t3[:, _EPG:, :]
        ztd = jnp.zeros((_BPAIR, _MAXSET - _EPG, _TD), _F32)
        mq = jnp.concatenate([tq, ztd], axis=1)
        mc = jnp.concatenate([tc, ztd], axis=1)

        la = dg(mq, mc, 2, 2) / jnp.float32(_SINK_TEMP)
        for _ in range(_SINK_ITERS):
            m2 = jnp.max(la, axis=2, keepdims=True)
            la = la - (m2 + jnp.log(jnp.sum(jnp.exp(la - m2), axis=2, keepdims=True)))
            m1 = jnp.max(la, axis=1, keepdims=True)
            la = la - (m1 + jnp.log(jnp.sum(jnp.exp(la - m1), axis=1, keepdims=True)))
        tmat = jnp.exp(la)

        if t < _NTIME - 1:
            qfc = dg(tmat[:, :_NPG, :_EPG], ec, 2, 1)
            cfq = dg(tmat[:, :_EPG, :_NPG], eq, 1, 1)
            inter = jnp.concatenate([qfc, cfq], axis=1).reshape(_BN, _ED)
        else:
            tmc = dg(tmat[:, :, :_EPG], tc, 2, 1)
            s2 = jnp.sum(jnp.abs(mq - tmc), axis=2)
            s1 = jnp.sum(s2, axis=1, keepdims=True)
            score_r[...] = jnp.broadcast_to(-s1, (_BPAIR, 128))


def _full(shape):
    nd = len(shape)
    return pl.BlockSpec(shape, lambda i: (0,) * nd)


def kernel(node_features, edge_features, from_idx, to_idx, graph_idx, params):
    p = params
    lf = from_idx.astype(jnp.int32).reshape(_G, 1, _BE)
    lt = to_idx.astype(jnp.int32).reshape(_G, 1, _BE)

    wc1 = jnp.pad(p["comb_W1"], ((0, 0), (0, _CH - 129)))
    bc1 = jnp.pad(p["comb_b1"], (0, _CH - 129)).reshape(1, _CH)
    wc2 = jnp.pad(p["comb_W2"], ((0, _CH - 129), (0, 0)))

    def b2(x):
        return x.reshape(1, -1)

    weights = (
        p["enc_node_W"], b2(p["enc_node_b"]), p["enc_edge_W"], b2(p["enc_edge_b"]),
        wc1, bc1, wc2, b2(p["comb_b2"]),
        p["msg_W1"], b2(p["msg_b1"]), p["msg_W2"], b2(p["msg_b2"]),
        p["rmsg_W1"], b2(p["rmsg_b1"]), p["rmsg_W2"], b2(p["rmsg_b2"]),
        p["upd_W1"], b2(p["upd_b1"]), p["upd_W2"], b2(p["upd_b2"]),
        p["t1_W"], b2(p["t1_b"]), p["t2_W"], b2(p["t2_b"]),
    )

    scores = pl.pallas_call(
        _main_kernel,
        grid=(_G,),
        in_specs=[
            pl.BlockSpec((_BN, _ND), lambda i: (i, 0)),
            pl.BlockSpec((_BE, _EIN), lambda i: (i, 0)),
            pl.BlockSpec((1, 1, _BE), lambda i: (i, 0, 0)),
            pl.BlockSpec((1, 1, _BE), lambda i: (i, 0, 0)),
        ] + [_full(w.shape) for w in weights],
        out_specs=pl.BlockSpec((_BPAIR, 128), lambda i: (i, 0)),
        out_shape=jax.ShapeDtypeStruct((_NPAIR, 128), _F32),
        compiler_params=pltpu.CompilerParams(
            dimension_semantics=("arbitrary",),
            vmem_limit_bytes=100 * 1024 * 1024),
    )(node_features, edge_features, lf, lt, *weights)

    return scores[:, 0]
